# Initial kernel scaffold; baseline (speedup 1.0000x reference)
import jax
import jax.numpy as jnp
from jax.experimental import pallas as pl


def kernel(P_coarse_b3n, P_fine_b3n, dP_coarse_b3n, F_skip_bcn, W1, g1, b1, W2, g2, b2, W3, b3):
    raise NotImplementedError("write your pallas kernel here")


# speedup vs baseline: 17.9110x; 17.9110x over previous
import jax
import jax.numpy as jnp
from jax.experimental import pallas as pl
from jax.experimental.pallas import tpu as pltpu

_B, _NC, _NF, _CSKIP, _H, _K, _G = 4, 2048, 8192, 128, 128, 3, 8
_T = 256
_NT = _NF // _T
_GN_N = (_H // _G) * _NF


def _dot(a, b):
    return jax.lax.dot_general(a, b, (((1,), (0,)), ((), ())),
                               preferred_element_type=jnp.float32)


def _silu(x):
    return x / (1.0 + jnp.exp(-x))


def _knn_body(pct_ref, pf_ref, dpc_ref, fs_ref, w1a_ref, w1b_ref,
              dp3_ref, h1_ref, s1_ref, q1_ref):
    t = pl.program_id(1)
    pc = pct_ref[0]
    pf = pf_ref[0]
    pp = pc[:, 0:1] * pc[:, 0:1] + pc[:, 1:2] * pc[:, 1:2] + pc[:, 2:3] * pc[:, 2:3]
    qq = pf[0:1] * pf[0:1] + pf[1:2] * pf[1:2] + pf[2:3] * pf[2:3]
    rb = lambda x: x.astype(jnp.bfloat16).astype(jnp.float32)
    pcb0, pcb1, pcb2 = rb(pc[:, 0:1]), rb(pc[:, 1:2]), rb(pc[:, 2:3])
    pfb0, pfb1, pfb2 = rb(pf[0:1]), rb(pf[1:2]), rb(pf[2:3])
    prod = pcb0 * pfb0 + pcb1 * pfb1 + pcb2 * pfb2
    d2 = (pp + qq) - 2.0 * prod

    iota = jax.lax.broadcasted_iota(jnp.int32, (_NC, _T), 0)
    wmat = jnp.zeros((_NC, _T), jnp.float32)
    wsum = jnp.zeros((1, _T), jnp.float32)
    for _ in range(_K):
        vk = jnp.min(d2, axis=0, keepdims=True)
        ik = jnp.min(jnp.where(d2 == vk, iota, _NC), axis=0, keepdims=True)
        sel = iota == ik
        wk = 1.0 / jnp.maximum(vk, 1e-12)
        wmat = wmat + jnp.where(sel, wk, 0.0)
        wsum = wsum + wk
        d2 = jnp.where(sel, jnp.inf, d2)
    wmat = wmat / wsum

    dp3 = _dot(dpc_ref[0], wmat)
    h1 = _dot(w1a_ref[...], dp3) + _dot(w1b_ref[...], fs_ref[0])
    dp3_ref[0] = dp3
    h1_ref[0] = h1

    @pl.when(t == 0)
    def _():
        s1_ref[...] = jnp.zeros_like(s1_ref)
        q1_ref[...] = jnp.zeros_like(q1_ref)

    s1_ref[0] += jnp.sum(h1, axis=1, keepdims=True)
    q1_ref[0] += jnp.sum(h1 * h1, axis=1, keepdims=True)


def _gn_affine(s_ref, q_ref, g_ref, b_ref):
    r = jax.lax.broadcasted_iota(jnp.int32, (_H, _H), 0) // (_H // _G)
    c = jax.lax.broadcasted_iota(jnp.int32, (_H, _H), 1) // (_H // _G)
    A = (r == c).astype(jnp.float32)
    mean = _dot(A, s_ref[0]) * (1.0 / _GN_N)
    var = _dot(A, q_ref[0]) * (1.0 / _GN_N) - mean * mean
    inv = jax.lax.rsqrt(var + 1e-5)
    a = g_ref[...] * inv
    return a, b_ref[...] - mean * a


def _mid_body(h1_ref, s1_ref, q1_ref, g_ref, b_ref, w2_ref,
              h2_ref, s2_ref, q2_ref):
    t = pl.program_id(1)
    a, c = _gn_affine(s1_ref, q1_ref, g_ref, b_ref)
    act = _silu(h1_ref[0] * a + c)
    h2 = _dot(w2_ref[...], act)
    h2_ref[0] = h2

    @pl.when(t == 0)
    def _():
        s2_ref[...] = jnp.zeros_like(s2_ref)
        q2_ref[...] = jnp.zeros_like(q2_ref)

    s2_ref[0] += jnp.sum(h2, axis=1, keepdims=True)
    q2_ref[0] += jnp.sum(h2 * h2, axis=1, keepdims=True)


def _out_body(h2_ref, s2_ref, q2_ref, g_ref, b_ref, w3_ref, b3_ref, dp3_ref,
              out_ref):
    a, c = _gn_affine(s2_ref, q2_ref, g_ref, b_ref)
    act = _silu(h2_ref[0] * a + c)
    out_ref[0] = dp3_ref[0] + _dot(w3_ref[...], act) + b3_ref[...]


def kernel(P_coarse_b3n, P_fine_b3n, dP_coarse_b3n, F_skip_bcn, W1, g1, b1,
           W2, g2, b2, W3, b3):
    f32 = jnp.float32
    pct = jnp.transpose(P_coarse_b3n, (0, 2, 1))
    w1a = W1[:, :3]
    w1b = W1[:, 3:]
    g1c, b1c = g1.reshape(_H, 1), b1.reshape(_H, 1)
    g2c, b2c = g2.reshape(_H, 1), b2.reshape(_H, 1)
    b3c = b3.reshape(3, 1)

    arb = pltpu.CompilerParams(
        dimension_semantics=("arbitrary", "arbitrary"))

    full = lambda shape: pl.BlockSpec(shape, lambda bi, ti: (0,) * len(shape))
    perb = lambda shape: pl.BlockSpec(shape, lambda bi, ti: (bi,) + (0,) * (len(shape) - 1))
    tile = lambda shape: pl.BlockSpec(shape, lambda bi, ti: (bi, 0, ti))

    dp3, h1, s1, q1 = pl.pallas_call(
        _knn_body,
        grid=(_B, _NT),
        in_specs=[perb((1, _NC, 3)), tile((1, 3, _T)), perb((1, 3, _NC)),
                  tile((1, _CSKIP, _T)), full((_H, 3)), full((_H, _CSKIP))],
        out_specs=[tile((1, 3, _T)), tile((1, _H, _T)),
                   perb((1, _H, 1)), perb((1, _H, 1))],
        out_shape=[jax.ShapeDtypeStruct((_B, 3, _NF), f32),
                   jax.ShapeDtypeStruct((_B, _H, _NF), f32),
                   jax.ShapeDtypeStruct((_B, _H, 1), f32),
                   jax.ShapeDtypeStruct((_B, _H, 1), f32)],
        compiler_params=arb,
    )(pct, P_fine_b3n, dP_coarse_b3n, F_skip_bcn, w1a, w1b)

    h2, s2, q2 = pl.pallas_call(
        _mid_body,
        grid=(_B, _NT),
        in_specs=[tile((1, _H, _T)), perb((1, _H, 1)), perb((1, _H, 1)),
                  full((_H, 1)), full((_H, 1)), full((_H, _H))],
        out_specs=[tile((1, _H, _T)), perb((1, _H, 1)), perb((1, _H, 1))],
        out_shape=[jax.ShapeDtypeStruct((_B, _H, _NF), f32),
                   jax.ShapeDtypeStruct((_B, _H, 1), f32),
                   jax.ShapeDtypeStruct((_B, _H, 1), f32)],
        compiler_params=arb,
    )(h1, s1, q1, g1c, b1c, W2)

    out = pl.pallas_call(
        _out_body,
        grid=(_B, _NT),
        in_specs=[tile((1, _H, _T)), perb((1, _H, 1)), perb((1, _H, 1)),
                  full((_H, 1)), full((_H, 1)), full((3, _H)), full((3, 1)),
                  tile((1, 3, _T))],
        out_specs=tile((1, 3, _T)),
        out_shape=jax.ShapeDtypeStruct((_B, 3, _NF), f32),
        compiler_params=arb,
    )(h2, s2, q2, g2c, b2c, W3, b3c, dp3)

    return out


# speedup vs baseline: 39.8152x; 2.2229x over previous
import jax
import jax.numpy as jnp
from jax.experimental import pallas as pl
from jax.experimental.pallas import tpu as pltpu

_B, _NC, _NF, _CSKIP, _H, _K, _G = 4, 2048, 8192, 128, 128, 3, 8
_T = 256
_NT = _NF // _T
_TM = 1024
_NTM = _NF // _TM
_GN_N = (_H // _G) * _NF
_KEEP = ~2047


def _dot(a, b):
    return jax.lax.dot_general(a, b, (((1,), (0,)), ((), ())),
                               preferred_element_type=jnp.float32)


def _silu(x):
    return x / (1.0 + jnp.exp(-x))


def _val(key):
    return jax.lax.bitcast_convert_type(
        jax.lax.bitcast_convert_type(key, jnp.int32) & _KEEP, jnp.float32)


def _knn_body(pct_ref, pcb_ref, pfb_ref, pf_ref, dpc_ref, fs_ref,
              w1a_ref, w1b_ref,
              dp3_ref, h1_ref, s1_ref, q1_ref, pp_ref):
    t = pl.program_id(1)

    @pl.when(t == 0)
    def _():
        pc = pct_ref[0]
        pp_ref[...] = (pc[:, 0:1] * pc[:, 0:1] + pc[:, 1:2] * pc[:, 1:2]
                       + pc[:, 2:3] * pc[:, 2:3])

    pf = pf_ref[0]
    qq = pf[0:1] * pf[0:1] + pf[1:2] * pf[1:2] + pf[2:3] * pf[2:3]
    prod = _dot(pcb_ref[0], pfb_ref[0])
    d2 = (pp_ref[...] - 2.0 * prod) + qq

    ibits = jax.lax.broadcasted_iota(jnp.int32, (_NC, _T), 0)
    key = jax.lax.bitcast_convert_type(
        (jax.lax.bitcast_convert_type(d2, jnp.int32) & _KEEP) | ibits,
        jnp.float32)
    inf = jnp.inf
    m1 = jnp.min(key, axis=0, keepdims=True)
    m2 = jnp.min(jnp.where(key > m1, key, inf), axis=0, keepdims=True)
    m3 = jnp.min(jnp.where(key > m2, key, inf), axis=0, keepdims=True)
    wsum = (1.0 / jnp.maximum(_val(m1), 1e-12)
            + 1.0 / jnp.maximum(_val(m2), 1e-12)
            + 1.0 / jnp.maximum(_val(m3), 1e-12))
    wmat = jnp.where(key <= m3,
                     1.0 / jnp.maximum(_val(key), 1e-12), 0.0)

    dp3 = _dot(dpc_ref[0], wmat) / wsum
    h1 = (_dot(w1a_ref[...], dp3) + _dot(w1b_ref[...], fs_ref[0]))
    dp3_ref[0] = dp3
    h1_ref[0] = h1

    @pl.when(t == 0)
    def _():
        s1_ref[...] = jnp.zeros_like(s1_ref)
        q1_ref[...] = jnp.zeros_like(q1_ref)

    s1_ref[0] += jnp.sum(h1, axis=1, keepdims=True)
    q1_ref[0] += jnp.sum(h1 * h1, axis=1, keepdims=True)


def _gn_affine(s_ref, q_ref, g_ref, b_ref):
    r = jax.lax.broadcasted_iota(jnp.int32, (_H, _H), 0) // (_H // _G)
    c = jax.lax.broadcasted_iota(jnp.int32, (_H, _H), 1) // (_H // _G)
    A = (r == c).astype(jnp.float32)
    mean = _dot(A, s_ref[0]) * (1.0 / _GN_N)
    var = _dot(A, q_ref[0]) * (1.0 / _GN_N) - mean * mean
    inv = jax.lax.rsqrt(var + 1e-5)
    a = g_ref[...] * inv
    return a, b_ref[...] - mean * a


def _mid_body(h1_ref, s1_ref, q1_ref, g_ref, b_ref, w2_ref,
              h2_ref, s2_ref, q2_ref):
    t = pl.program_id(1)
    a, c = _gn_affine(s1_ref, q1_ref, g_ref, b_ref)
    act = _silu(h1_ref[0] * a + c)
    h2 = _dot(w2_ref[...], act)
    h2_ref[0] = h2

    @pl.when(t == 0)
    def _():
        s2_ref[...] = jnp.zeros_like(s2_ref)
        q2_ref[...] = jnp.zeros_like(q2_ref)

    s2_ref[0] += jnp.sum(h2, axis=1, keepdims=True)
    q2_ref[0] += jnp.sum(h2 * h2, axis=1, keepdims=True)


def _out_body(h2_ref, s2_ref, q2_ref, g_ref, b_ref, w3_ref, b3_ref, dp3_ref,
              out_ref):
    a, c = _gn_affine(s2_ref, q2_ref, g_ref, b_ref)
    act = _silu(h2_ref[0] * a + c)
    out_ref[0] = dp3_ref[0] + _dot(w3_ref[...], act) + b3_ref[...]


def kernel(P_coarse_b3n, P_fine_b3n, dP_coarse_b3n, F_skip_bcn, W1, g1, b1,
           W2, g2, b2, W3, b3):
    f32 = jnp.float32
    bf16 = jnp.bfloat16
    pct = jnp.transpose(P_coarse_b3n, (0, 2, 1))
    pcb = pct.astype(bf16)
    pfb = P_fine_b3n.astype(bf16)
    w1a = W1[:, :3]
    w1b = W1[:, 3:]
    g1c, b1c = g1.reshape(_H, 1), b1.reshape(_H, 1)
    g2c, b2c = g2.reshape(_H, 1), b2.reshape(_H, 1)
    b3c = b3.reshape(3, 1)

    arb = pltpu.CompilerParams(
        dimension_semantics=("arbitrary", "arbitrary"))

    full = lambda shape: pl.BlockSpec(shape, lambda bi, ti: (0,) * len(shape))
    perb = lambda shape: pl.BlockSpec(shape, lambda bi, ti: (bi,) + (0,) * (len(shape) - 1))
    tile = lambda shape: pl.BlockSpec(shape, lambda bi, ti: (bi, 0, ti))

    dp3, h1, s1, q1 = pl.pallas_call(
        _knn_body,
        grid=(_B, _NT),
        in_specs=[perb((1, _NC, 3)), perb((1, _NC, 3)), tile((1, 3, _T)),
                  tile((1, 3, _T)), perb((1, 3, _NC)), tile((1, _CSKIP, _T)),
                  full((_H, 3)), full((_H, _CSKIP))],
        out_specs=[tile((1, 3, _T)), tile((1, _H, _T)),
                   perb((1, _H, 1)), perb((1, _H, 1))],
        out_shape=[jax.ShapeDtypeStruct((_B, 3, _NF), f32),
                   jax.ShapeDtypeStruct((_B, _H, _NF), f32),
                   jax.ShapeDtypeStruct((_B, _H, 1), f32),
                   jax.ShapeDtypeStruct((_B, _H, 1), f32)],
        scratch_shapes=[pltpu.VMEM((_NC, 1), f32)],
        compiler_params=arb,
    )(pct, pcb, pfb, P_fine_b3n, dP_coarse_b3n, F_skip_bcn, w1a, w1b)

    tilem = lambda shape: pl.BlockSpec(shape, lambda bi, ti: (bi, 0, ti))

    h2, s2, q2 = pl.pallas_call(
        _mid_body,
        grid=(_B, _NTM),
        in_specs=[tilem((1, _H, _TM)), perb((1, _H, 1)), perb((1, _H, 1)),
                  full((_H, 1)), full((_H, 1)), full((_H, _H))],
        out_specs=[tilem((1, _H, _TM)), perb((1, _H, 1)), perb((1, _H, 1))],
        out_shape=[jax.ShapeDtypeStruct((_B, _H, _NF), f32),
                   jax.ShapeDtypeStruct((_B, _H, 1), f32),
                   jax.ShapeDtypeStruct((_B, _H, 1), f32)],
        compiler_params=arb,
    )(h1, s1, q1, g1c, b1c, W2)

    out = pl.pallas_call(
        _out_body,
        grid=(_B, _NTM),
        in_specs=[tilem((1, _H, _TM)), perb((1, _H, 1)), perb((1, _H, 1)),
                  full((_H, 1)), full((_H, 1)), full((3, _H)), full((3, 1)),
                  tilem((1, 3, _TM))],
        out_specs=tilem((1, 3, _TM)),
        out_shape=jax.ShapeDtypeStruct((_B, 3, _NF), f32),
        compiler_params=arb,
    )(h2, s2, q2, g2c, b2c, W3, b3c, dp3)

    return out


# speedup vs baseline: 48.6263x; 1.2213x over previous
import jax
import jax.numpy as jnp
from jax.experimental import pallas as pl
from jax.experimental.pallas import tpu as pltpu

_B, _NC, _NF, _CSKIP, _H, _K, _G = 4, 2048, 8192, 128, 128, 3, 8
_T = 512
_NT = _NF // _T
_TM = 1024
_NTM = _NF // _TM
_GN_N = (_H // _G) * _NF
_KEEP = ~2047


def _dot(a, b):
    return jax.lax.dot_general(a, b, (((1,), (0,)), ((), ())),
                               preferred_element_type=jnp.float32)


def _dotb(a, b):
    return _dot(a.astype(jnp.bfloat16), b.astype(jnp.bfloat16))


def _silu(x):
    return x / (1.0 + jnp.exp(-x))


def _knn_body(pct_ref, pcb_ref, pfb_ref, pf_ref, dpc_ref, fs_ref,
              w1a_ref, w1b_ref,
              dp3_ref, h1_ref, s1_ref, q1_ref, pp_ref):
    t = pl.program_id(1)

    @pl.when(t == 0)
    def _():
        pc = pct_ref[0]
        pp_ref[...] = (pc[:, 0:1] * pc[:, 0:1] + pc[:, 1:2] * pc[:, 1:2]
                       + pc[:, 2:3] * pc[:, 2:3])

    pf = pf_ref[0]
    qq = pf[0:1] * pf[0:1] + pf[1:2] * pf[1:2] + pf[2:3] * pf[2:3]
    prod = _dot(pcb_ref[0], pfb_ref[0])
    d2 = (pp_ref[...] - 2.0 * prod) + qq

    ibits = jax.lax.broadcasted_iota(jnp.int32, (_NC, _T), 0)
    key = jax.lax.bitcast_convert_type(
        (jax.lax.bitcast_convert_type(d2, jnp.int32) & _KEEP) | ibits,
        jnp.float32)
    inf = jnp.inf
    m1 = jnp.min(key, axis=0, keepdims=True)
    m2 = jnp.min(jnp.where(key > m1, key, inf), axis=0, keepdims=True)
    m3 = jnp.min(jnp.where(key > m2, key, inf), axis=0, keepdims=True)
    wsum = (1.0 / jnp.maximum(m1, 1e-12)
            + 1.0 / jnp.maximum(m2, 1e-12)
            + 1.0 / jnp.maximum(m3, 1e-12))
    wmat = jnp.where(key <= m3,
                     1.0 / jnp.maximum(key, 1e-12), 0.0).astype(jnp.bfloat16)

    dp3 = _dot(dpc_ref[0], wmat) / wsum
    h1 = _dotb(w1a_ref[...], dp3) + _dot(w1b_ref[...], fs_ref[0])
    dp3_ref[0] = dp3
    h1_ref[0] = h1.astype(jnp.bfloat16)

    @pl.when(t == 0)
    def _():
        s1_ref[...] = jnp.zeros_like(s1_ref)
        q1_ref[...] = jnp.zeros_like(q1_ref)

    s1_ref[0] += jnp.sum(h1, axis=1, keepdims=True)
    q1_ref[0] += jnp.sum(h1 * h1, axis=1, keepdims=True)


def _gn_affine(s_ref, q_ref, g_ref, b_ref):
    r = jax.lax.broadcasted_iota(jnp.int32, (_H, _H), 0) // (_H // _G)
    c = jax.lax.broadcasted_iota(jnp.int32, (_H, _H), 1) // (_H // _G)
    A = (r == c).astype(jnp.float32)
    mean = _dot(A, s_ref[0]) * (1.0 / _GN_N)
    var = _dot(A, q_ref[0]) * (1.0 / _GN_N) - mean * mean
    inv = jax.lax.rsqrt(var + 1e-5)
    a = g_ref[...] * inv
    return a, b_ref[...] - mean * a


def _mid_body(h1_ref, s1_ref, q1_ref, g_ref, b_ref, w2_ref,
              h2_ref, s2_ref, q2_ref):
    t = pl.program_id(1)
    a, c = _gn_affine(s1_ref, q1_ref, g_ref, b_ref)
    act = _silu(h1_ref[0].astype(jnp.float32) * a + c)
    h2 = _dotb(w2_ref[...], act)
    h2_ref[0] = h2.astype(jnp.bfloat16)

    @pl.when(t == 0)
    def _():
        s2_ref[...] = jnp.zeros_like(s2_ref)
        q2_ref[...] = jnp.zeros_like(q2_ref)

    s2_ref[0] += jnp.sum(h2, axis=1, keepdims=True)
    q2_ref[0] += jnp.sum(h2 * h2, axis=1, keepdims=True)


def _out_body(h2_ref, s2_ref, q2_ref, g_ref, b_ref, w3_ref, b3_ref, dp3_ref,
              out_ref):
    a, c = _gn_affine(s2_ref, q2_ref, g_ref, b_ref)
    act = _silu(h2_ref[0].astype(jnp.float32) * a + c)
    out_ref[0] = dp3_ref[0] + _dotb(w3_ref[...], act) + b3_ref[...]


def kernel(P_coarse_b3n, P_fine_b3n, dP_coarse_b3n, F_skip_bcn, W1, g1, b1,
           W2, g2, b2, W3, b3):
    f32 = jnp.float32
    bf16 = jnp.bfloat16
    pct = jnp.transpose(P_coarse_b3n, (0, 2, 1))
    pcb = pct.astype(bf16)
    pfb = P_fine_b3n.astype(bf16)
    dpcb = dP_coarse_b3n.astype(bf16)
    fsb = F_skip_bcn.astype(bf16)
    w1a = W1[:, :3]
    w1b = W1[:, 3:].astype(bf16)
    g1c, b1c = g1.reshape(_H, 1), b1.reshape(_H, 1)
    g2c, b2c = g2.reshape(_H, 1), b2.reshape(_H, 1)
    b3c = b3.reshape(3, 1)

    arb = pltpu.CompilerParams(
        dimension_semantics=("arbitrary", "arbitrary"))

    full = lambda shape: pl.BlockSpec(shape, lambda bi, ti: (0,) * len(shape))
    perb = lambda shape: pl.BlockSpec(shape, lambda bi, ti: (bi,) + (0,) * (len(shape) - 1))
    tile = lambda shape: pl.BlockSpec(shape, lambda bi, ti: (bi, 0, ti))

    dp3, h1, s1, q1 = pl.pallas_call(
        _knn_body,
        grid=(_B, _NT),
        in_specs=[perb((1, _NC, 3)), perb((1, _NC, 3)), tile((1, 3, _T)),
                  tile((1, 3, _T)), perb((1, 3, _NC)), tile((1, _CSKIP, _T)),
                  full((_H, 3)), full((_H, _CSKIP))],
        out_specs=[tile((1, 3, _T)), tile((1, _H, _T)),
                   perb((1, _H, 1)), perb((1, _H, 1))],
        out_shape=[jax.ShapeDtypeStruct((_B, 3, _NF), f32),
                   jax.ShapeDtypeStruct((_B, _H, _NF), bf16),
                   jax.ShapeDtypeStruct((_B, _H, 1), f32),
                   jax.ShapeDtypeStruct((_B, _H, 1), f32)],
        scratch_shapes=[pltpu.VMEM((_NC, 1), f32)],
        compiler_params=arb,
    )(pct, pcb, pfb, P_fine_b3n, dpcb, fsb, w1a, w1b)

    h2, s2, q2 = pl.pallas_call(
        _mid_body,
        grid=(_B, _NTM),
        in_specs=[tile((1, _H, _TM)), perb((1, _H, 1)), perb((1, _H, 1)),
                  full((_H, 1)), full((_H, 1)), full((_H, _H))],
        out_specs=[tile((1, _H, _TM)), perb((1, _H, 1)), perb((1, _H, 1))],
        out_shape=[jax.ShapeDtypeStruct((_B, _H, _NF), bf16),
                   jax.ShapeDtypeStruct((_B, _H, 1), f32),
                   jax.ShapeDtypeStruct((_B, _H, 1), f32)],
        compiler_params=arb,
    )(h1, s1, q1, g1c, b1c, W2)

    out = pl.pallas_call(
        _out_body,
        grid=(_B, _NTM),
        in_specs=[tile((1, _H, _TM)), perb((1, _H, 1)), perb((1, _H, 1)),
                  full((_H, 1)), full((_H, 1)), full((3, _H)), full((3, 1)),
                  tile((1, 3, _TM))],
        out_specs=tile((1, 3, _TM)),
        out_shape=jax.ShapeDtypeStruct((_B, 3, _NF), f32),
        compiler_params=arb,
    )(h2, s2, q2, g2c, b2c, W3, b3c, dp3)

    return out
